# baseline (speedup 1.0000x reference)
import functools

import jax
import jax.numpy as jnp
from jax.experimental import pallas as pl
from jax.experimental.pallas import tpu as pltpu

_ROWS = 2560
_CHUNK = 64
_NCHUNK_PAD = 48
_S_PAD = 264
_GATHER = 3


def _body(cidx_ref, rmod_ref, e_ref, x_ref, *rest, num_blocks, rows, n_seg):
    g_refs = rest[:_GATHER]
    w1_ref, b1_ref, w2_ref, b2_ref, w3_ref, b3_ref, out_ref, acc_ref, p_ref = \
        rest[_GATHER:]
    b = pl.program_id(0)

    @pl.when(b == 0)
    def _build():
        col = jax.lax.broadcasted_iota(jnp.int32, (_NCHUNK_PAD, rows), 1)
        row = jax.lax.broadcasted_iota(jnp.int32, (_NCHUNK_PAD, rows), 0)
        p_ref[...] = ((col // _CHUNK) == row).astype(jnp.bfloat16)
        acc_ref[...] = jnp.zeros_like(acc_ref)

    xall = jnp.concatenate([x_ref[...]] + [g[...] for g in g_refs], axis=0)
    a = jnp.dot(xall, w1_ref[...], preferred_element_type=jnp.float32)
    a = jnp.maximum(a + b1_ref[...], 0.0)

    csum = jnp.dot(p_ref[...], a[0:rows, :].astype(jnp.bfloat16),
                   preferred_element_type=jnp.float32)
    cstart = b * rows + _CHUNK * jax.lax.broadcasted_iota(
        jnp.int32, (1, _NCHUNK_PAD), 1)
    stair = (cstart >= e_ref[...]).astype(jnp.float32)
    acc_ref[...] += jnp.dot(stair, csum, preferred_element_type=jnp.float32)

    riota = jax.lax.broadcasted_iota(jnp.int32, (_CHUNK, 1), 0)
    corrs = []
    for k in range(_GATHER):
        t = jnp.clip(b * _GATHER + k, 0, rmod_ref.shape[0] - 1)
        rmod = rmod_ref[t]
        mask = (riota >= rmod) & (rmod > 0)
        rows_k = a[rows + k * _CHUNK:rows + (k + 1) * _CHUNK, :]
        corrs.append(jnp.sum(jnp.where(mask, rows_k, 0.0), axis=0,
                             keepdims=True))
    start = jnp.minimum(b * _GATHER, _S_PAD - _GATHER)
    acc_ref[pl.ds(start, _GATHER), :] += jnp.concatenate(corrs, axis=0)

    @pl.when(b == num_blocks - 1)
    def _finalize():
        seg = acc_ref[0:n_seg, :] - acc_ref[1:n_seg + 1, :]
        d = e_ref[1:n_seg + 1, :] - e_ref[0:n_seg, :]
        counts = jnp.maximum(d.astype(jnp.float32), 1.0)
        mean = seg / counts
        h2 = jnp.dot(mean, w2_ref[...], preferred_element_type=jnp.float32) + b2_ref[...]
        out_ref[...] = jnp.dot(h2, w3_ref[...], preferred_element_type=jnp.float32) + b3_ref[...]


def kernel(x1, edge_slices, W1, b1, W2, b2, W3, b3):
    n, d_in = x1.shape
    d_out = W2.shape[1]
    n_seg = edge_slices.shape[0] - 1
    rows = _ROWS
    num_blocks = n // rows
    assert num_blocks * rows == n and rows % _CHUNK == 0

    n_bound = num_blocks * _GATHER
    e_pad = jnp.concatenate(
        [edge_slices,
         jnp.full((_S_PAD - edge_slices.shape[0],), n, dtype=jnp.int32)])
    e_gather = jnp.concatenate(
        [edge_slices,
         jnp.full((n_bound - edge_slices.shape[0],), n, dtype=jnp.int32)])
    chunk_idx = jnp.clip(e_gather // _CHUNK, 0, n // _CHUNK - 1).astype(jnp.int32)
    rmod = (e_gather % _CHUNK).astype(jnp.int32)

    body = functools.partial(_body, num_blocks=num_blocks, rows=rows,
                             n_seg=n_seg)
    full = lambda shape: pl.BlockSpec(shape, lambda b, ci, rm: (0, 0))

    def _gather_spec(k):
        def imap(b, ci, rm):
            t = jnp.clip(b * _GATHER + k, 0, n_bound - 1)
            return (ci[t], 0)
        return pl.BlockSpec((_CHUNK, d_in), imap)

    grid_spec = pltpu.PrefetchScalarGridSpec(
        num_scalar_prefetch=2,
        grid=(num_blocks,),
        in_specs=[
            full((_S_PAD, 1)),
            pl.BlockSpec((rows, d_in), lambda b, ci, rm: (b, 0)),
            *[_gather_spec(k) for k in range(_GATHER)],
            full((d_in, d_out)),
            full((1, d_out)),
            full((d_out, d_out)),
            full((1, d_out)),
            full((d_out, d_out)),
            full((1, d_out)),
        ],
        out_specs=full((n_seg, d_out)),
        scratch_shapes=[
            pltpu.VMEM((_S_PAD, d_out), jnp.float32),
            pltpu.VMEM((_NCHUNK_PAD, rows), jnp.bfloat16),
        ],
    )
    out = pl.pallas_call(
        body,
        grid_spec=grid_spec,
        out_shape=jax.ShapeDtypeStruct((n_seg, d_out), jnp.float32),
        compiler_params=pltpu.CompilerParams(
            dimension_semantics=("arbitrary",),
        ),
    )(chunk_idx, rmod, e_pad.reshape(_S_PAD, 1), x1,
      *([x1] * _GATHER), W1, b1.reshape(1, d_out), W2, b2.reshape(1, d_out),
      W3, b3.reshape(1, d_out))
    return out


# speedup vs baseline: 1.1009x; 1.1009x over previous
import functools

import jax
import jax.numpy as jnp
from jax.experimental import pallas as pl
from jax.experimental.pallas import tpu as pltpu

_ROWS = 2560
_CHUNK = 64
_NCHUNK_PAD = 48
_S_PAD = 264
_GATHER = 3


def _body(cidx_ref, rmod_ref, e_ref, x_ref, *rest, num_blocks, rows, n_seg):
    g_refs = rest[:_GATHER]
    w1_ref, b1_ref, w2_ref, b2_ref, w3_ref, b3_ref, out_ref, acc_ref = \
        rest[_GATHER:]
    b = pl.program_id(0)

    @pl.when(b == 0)
    def _build():
        acc_ref[...] = jnp.zeros_like(acc_ref)

    xall = jnp.concatenate([x_ref[...]] + [g[...] for g in g_refs], axis=0)
    a = jnp.dot(xall, w1_ref[...], preferred_element_type=jnp.float32)
    a = jnp.maximum(a + b1_ref[...], 0.0)

    csum40 = jnp.sum(a[0:rows, :].reshape(rows // _CHUNK, _CHUNK, -1), axis=1)
    csum = jnp.concatenate(
        [csum40, jnp.zeros((_NCHUNK_PAD - rows // _CHUNK, csum40.shape[1]),
                           jnp.float32)], axis=0)
    cstart = b * rows + _CHUNK * jax.lax.broadcasted_iota(
        jnp.int32, (1, _NCHUNK_PAD), 1)
    stair = (cstart >= e_ref[...]).astype(jnp.float32)
    acc_ref[...] += jnp.dot(stair, csum, preferred_element_type=jnp.float32)

    riota = jax.lax.broadcasted_iota(jnp.int32, (_CHUNK, 1), 0)
    corrs = []
    for k in range(_GATHER):
        t = jnp.clip(b * _GATHER + k, 0, rmod_ref.shape[0] - 1)
        rmod = rmod_ref[t]
        mask = (riota >= rmod) & (rmod > 0)
        rows_k = a[rows + k * _CHUNK:rows + (k + 1) * _CHUNK, :]
        corrs.append(jnp.sum(jnp.where(mask, rows_k, 0.0), axis=0,
                             keepdims=True))
    start = jnp.minimum(b * _GATHER, _S_PAD - _GATHER)
    acc_ref[pl.ds(start, _GATHER), :] += jnp.concatenate(corrs, axis=0)

    @pl.when(b == num_blocks - 1)
    def _finalize():
        seg = acc_ref[0:n_seg, :] - acc_ref[1:n_seg + 1, :]
        d = e_ref[1:n_seg + 1, :] - e_ref[0:n_seg, :]
        counts = jnp.maximum(d.astype(jnp.float32), 1.0)
        mean = seg / counts
        h2 = jnp.dot(mean, w2_ref[...], preferred_element_type=jnp.float32) + b2_ref[...]
        out_ref[...] = jnp.dot(h2, w3_ref[...], preferred_element_type=jnp.float32) + b3_ref[...]


def kernel(x1, edge_slices, W1, b1, W2, b2, W3, b3):
    n, d_in = x1.shape
    d_out = W2.shape[1]
    n_seg = edge_slices.shape[0] - 1
    rows = _ROWS
    num_blocks = n // rows
    assert num_blocks * rows == n and rows % _CHUNK == 0

    n_bound = num_blocks * _GATHER
    e_pad = jnp.concatenate(
        [edge_slices,
         jnp.full((_S_PAD - edge_slices.shape[0],), n, dtype=jnp.int32)])
    e_gather = jnp.concatenate(
        [edge_slices,
         jnp.full((n_bound - edge_slices.shape[0],), n, dtype=jnp.int32)])
    chunk_idx = jnp.clip(e_gather // _CHUNK, 0, n // _CHUNK - 1).astype(jnp.int32)
    rmod = (e_gather % _CHUNK).astype(jnp.int32)

    body = functools.partial(_body, num_blocks=num_blocks, rows=rows,
                             n_seg=n_seg)
    full = lambda shape: pl.BlockSpec(shape, lambda b, ci, rm: (0, 0))

    def _gather_spec(k):
        def imap(b, ci, rm):
            t = jnp.clip(b * _GATHER + k, 0, n_bound - 1)
            return (ci[t], 0)
        return pl.BlockSpec((_CHUNK, d_in), imap)

    grid_spec = pltpu.PrefetchScalarGridSpec(
        num_scalar_prefetch=2,
        grid=(num_blocks,),
        in_specs=[
            full((_S_PAD, 1)),
            pl.BlockSpec((rows, d_in), lambda b, ci, rm: (b, 0)),
            *[_gather_spec(k) for k in range(_GATHER)],
            full((d_in, d_out)),
            full((1, d_out)),
            full((d_out, d_out)),
            full((1, d_out)),
            full((d_out, d_out)),
            full((1, d_out)),
        ],
        out_specs=full((n_seg, d_out)),
        scratch_shapes=[
            pltpu.VMEM((_S_PAD, d_out), jnp.float32),
        ],
    )
    out = pl.pallas_call(
        body,
        grid_spec=grid_spec,
        out_shape=jax.ShapeDtypeStruct((n_seg, d_out), jnp.float32),
        compiler_params=pltpu.CompilerParams(
            dimension_semantics=("arbitrary",),
        ),
    )(chunk_idx, rmod, e_pad.reshape(_S_PAD, 1), x1,
      *([x1] * _GATHER), W1, b1.reshape(1, d_out), W2, b2.reshape(1, d_out),
      W3, b3.reshape(1, d_out))
    return out


# speedup vs baseline: 1.7553x; 1.5944x over previous
import functools

import jax
import jax.numpy as jnp
from jax.experimental import pallas as pl
from jax.experimental.pallas import tpu as pltpu

_ROWS = 6400
_CHUNK = 64
_NCHUNK_PAD = 104
_S_PAD = 264
_GATHER = 6


def _body(cidx_ref, rmod_ref, e_ref, x_ref, *rest, num_blocks, rows, n_seg):
    g_refs = rest[:_GATHER]
    w1_ref, b1_ref, w2_ref, b2_ref, w3_ref, b3_ref, out_ref, acc_ref = \
        rest[_GATHER:]
    b = pl.program_id(0)

    @pl.when(b == 0)
    def _build():
        acc_ref[...] = jnp.zeros_like(acc_ref)

    a = jnp.dot(x_ref[...], w1_ref[...], preferred_element_type=jnp.float32)
    a = jnp.maximum(a + b1_ref[...], 0.0)
    xg = jnp.concatenate([g[...] for g in g_refs], axis=0)
    ag = jnp.dot(xg, w1_ref[...], preferred_element_type=jnp.float32)
    ag = jnp.maximum(ag + b1_ref[...], 0.0)

    csum_n = jnp.sum(a.reshape(rows // _CHUNK, _CHUNK, -1), axis=1)
    csum = jnp.concatenate(
        [csum_n, jnp.zeros((_NCHUNK_PAD - rows // _CHUNK, csum_n.shape[1]),
                           jnp.float32)], axis=0)
    cstart = b * rows + _CHUNK * jax.lax.broadcasted_iota(
        jnp.int32, (1, _NCHUNK_PAD), 1)
    stair = (cstart >= e_ref[...]).astype(jnp.float32)
    acc_ref[...] += jnp.dot(stair, csum, preferred_element_type=jnp.float32)

    riota = jax.lax.broadcasted_iota(jnp.int32, (_CHUNK, 1), 0)
    corrs = []
    for k in range(_GATHER):
        t = jnp.clip(b * _GATHER + k, 0, rmod_ref.shape[0] - 1)
        rmod = rmod_ref[t]
        mask = (riota >= rmod) & (rmod > 0)
        rows_k = ag[k * _CHUNK:(k + 1) * _CHUNK, :]
        corrs.append(jnp.sum(jnp.where(mask, rows_k, 0.0), axis=0,
                             keepdims=True))
    start = jnp.minimum(b * _GATHER, _S_PAD - _GATHER)
    acc_ref[pl.ds(start, _GATHER), :] += jnp.concatenate(corrs, axis=0)

    @pl.when(b == num_blocks - 1)
    def _finalize():
        seg = acc_ref[0:n_seg, :] - acc_ref[1:n_seg + 1, :]
        d = e_ref[1:n_seg + 1, :] - e_ref[0:n_seg, :]
        counts = jnp.maximum(d.astype(jnp.float32), 1.0)
        mean = seg / counts
        h2 = jnp.dot(mean, w2_ref[...], preferred_element_type=jnp.float32) + b2_ref[...]
        out_ref[...] = jnp.dot(h2, w3_ref[...], preferred_element_type=jnp.float32) + b3_ref[...]


def kernel(x1, edge_slices, W1, b1, W2, b2, W3, b3):
    n, d_in = x1.shape
    d_out = W2.shape[1]
    n_seg = edge_slices.shape[0] - 1
    rows = _ROWS
    num_blocks = n // rows
    assert num_blocks * rows == n and rows % _CHUNK == 0

    n_bound = num_blocks * _GATHER
    e_pad = jnp.concatenate(
        [edge_slices,
         jnp.full((_S_PAD - edge_slices.shape[0],), n, dtype=jnp.int32)])
    e_gather = jnp.concatenate(
        [edge_slices,
         jnp.full((n_bound - edge_slices.shape[0],), n, dtype=jnp.int32)])
    chunk_idx = jnp.clip(e_gather // _CHUNK, 0, n // _CHUNK - 1).astype(jnp.int32)
    rmod = (e_gather % _CHUNK).astype(jnp.int32)

    body = functools.partial(_body, num_blocks=num_blocks, rows=rows,
                             n_seg=n_seg)
    full = lambda shape: pl.BlockSpec(shape, lambda b, ci, rm: (0, 0))

    def _gather_spec(k):
        def imap(b, ci, rm):
            t = jnp.clip(b * _GATHER + k, 0, n_bound - 1)
            return (ci[t], 0)
        return pl.BlockSpec((_CHUNK, d_in), imap)

    grid_spec = pltpu.PrefetchScalarGridSpec(
        num_scalar_prefetch=2,
        grid=(num_blocks,),
        in_specs=[
            full((_S_PAD, 1)),
            pl.BlockSpec((rows, d_in), lambda b, ci, rm: (b, 0)),
            *[_gather_spec(k) for k in range(_GATHER)],
            full((d_in, d_out)),
            full((1, d_out)),
            full((d_out, d_out)),
            full((1, d_out)),
            full((d_out, d_out)),
            full((1, d_out)),
        ],
        out_specs=full((n_seg, d_out)),
        scratch_shapes=[
            pltpu.VMEM((_S_PAD, d_out), jnp.float32),
        ],
    )
    out = pl.pallas_call(
        body,
        grid_spec=grid_spec,
        out_shape=jax.ShapeDtypeStruct((n_seg, d_out), jnp.float32),
        compiler_params=pltpu.CompilerParams(
            dimension_semantics=("arbitrary",),
        ),
    )(chunk_idx, rmod, e_pad.reshape(_S_PAD, 1), x1,
      *([x1] * _GATHER), W1, b1.reshape(1, d_out), W2, b2.reshape(1, d_out),
      W3, b3.reshape(1, d_out))
    return out


# speedup vs baseline: 2.3362x; 1.3310x over previous
import functools

import jax
import jax.numpy as jnp
from jax.experimental import pallas as pl
from jax.experimental.pallas import tpu as pltpu

_ROWS = 16000
_CHUNK = 64
_NCHUNK_PAD = 256
_S_PAD = 264
_GATHER = 13


def _body(cidx_ref, rmod_ref, e_ref, x_ref, *rest, num_blocks, rows, n_seg):
    g_refs = rest[:_GATHER]
    w1_ref, b1_ref, w2_ref, b2_ref, w3_ref, b3_ref, out_ref, acc_ref = \
        rest[_GATHER:]
    b = pl.program_id(0)

    @pl.when(b == 0)
    def _build():
        acc_ref[...] = jnp.zeros_like(acc_ref)

    a = jnp.dot(x_ref[...], w1_ref[...], preferred_element_type=jnp.float32)
    a = jnp.maximum(a + b1_ref[...], 0.0)
    xg = jnp.concatenate([g[...] for g in g_refs], axis=0)
    ag = jnp.dot(xg, w1_ref[...], preferred_element_type=jnp.float32)
    ag = jnp.maximum(ag + b1_ref[...], 0.0)

    csum_n = jnp.sum(a.reshape(rows // _CHUNK, _CHUNK, -1), axis=1)
    csum = jnp.concatenate(
        [csum_n, jnp.zeros((_NCHUNK_PAD - rows // _CHUNK, csum_n.shape[1]),
                           jnp.float32)], axis=0)
    cstart = b * rows + _CHUNK * jax.lax.broadcasted_iota(
        jnp.int32, (1, _NCHUNK_PAD), 1)
    stair = (cstart >= e_ref[...]).astype(jnp.float32)
    acc_ref[...] += jnp.dot(stair, csum, preferred_element_type=jnp.float32)

    riota = jax.lax.broadcasted_iota(jnp.int32, (_CHUNK, 1), 0)
    corrs = []
    for k in range(_GATHER):
        t = jnp.clip(b * _GATHER + k, 0, rmod_ref.shape[0] - 1)
        rmod = rmod_ref[t]
        mask = (riota >= rmod) & (rmod > 0)
        rows_k = ag[k * _CHUNK:(k + 1) * _CHUNK, :]
        corrs.append(jnp.sum(jnp.where(mask, rows_k, 0.0), axis=0,
                             keepdims=True))
    start = jnp.minimum(b * _GATHER, _S_PAD - _GATHER)
    acc_ref[pl.ds(start, _GATHER), :] += jnp.concatenate(corrs, axis=0)

    @pl.when(b == num_blocks - 1)
    def _finalize():
        seg = acc_ref[0:n_seg, :] - acc_ref[1:n_seg + 1, :]
        d = e_ref[1:n_seg + 1, :] - e_ref[0:n_seg, :]
        counts = jnp.maximum(d.astype(jnp.float32), 1.0)
        mean = seg / counts
        h2 = jnp.dot(mean, w2_ref[...], preferred_element_type=jnp.float32) + b2_ref[...]
        out_ref[...] = jnp.dot(h2, w3_ref[...], preferred_element_type=jnp.float32) + b3_ref[...]


def kernel(x1, edge_slices, W1, b1, W2, b2, W3, b3):
    n, d_in = x1.shape
    d_out = W2.shape[1]
    n_seg = edge_slices.shape[0] - 1
    rows = _ROWS
    num_blocks = n // rows
    assert num_blocks * rows == n and rows % _CHUNK == 0

    n_bound = num_blocks * _GATHER
    e_pad = jnp.concatenate(
        [edge_slices,
         jnp.full((_S_PAD - edge_slices.shape[0],), n, dtype=jnp.int32)])
    e_gather = jnp.concatenate(
        [edge_slices,
         jnp.full((n_bound - edge_slices.shape[0],), n, dtype=jnp.int32)])
    chunk_idx = jnp.clip(e_gather // _CHUNK, 0, n // _CHUNK - 1).astype(jnp.int32)
    rmod = (e_gather % _CHUNK).astype(jnp.int32)

    body = functools.partial(_body, num_blocks=num_blocks, rows=rows,
                             n_seg=n_seg)
    full = lambda shape: pl.BlockSpec(shape, lambda b, ci, rm: (0, 0))

    def _gather_spec(k):
        def imap(b, ci, rm):
            t = jnp.clip(b * _GATHER + k, 0, n_bound - 1)
            return (ci[t], 0)
        return pl.BlockSpec((_CHUNK, d_in), imap)

    grid_spec = pltpu.PrefetchScalarGridSpec(
        num_scalar_prefetch=2,
        grid=(num_blocks,),
        in_specs=[
            full((_S_PAD, 1)),
            pl.BlockSpec((rows, d_in), lambda b, ci, rm: (b, 0)),
            *[_gather_spec(k) for k in range(_GATHER)],
            full((d_in, d_out)),
            full((1, d_out)),
            full((d_out, d_out)),
            full((1, d_out)),
            full((d_out, d_out)),
            full((1, d_out)),
        ],
        out_specs=full((n_seg, d_out)),
        scratch_shapes=[
            pltpu.VMEM((_S_PAD, d_out), jnp.float32),
        ],
    )
    out = pl.pallas_call(
        body,
        grid_spec=grid_spec,
        out_shape=jax.ShapeDtypeStruct((n_seg, d_out), jnp.float32),
        compiler_params=pltpu.CompilerParams(
            dimension_semantics=("arbitrary",),
        ),
    )(chunk_idx, rmod, e_pad.reshape(_S_PAD, 1), x1,
      *([x1] * _GATHER), W1, b1.reshape(1, d_out), W2, b2.reshape(1, d_out),
      W3, b3.reshape(1, d_out))
    return out


# speedup vs baseline: 2.3808x; 1.0191x over previous
import functools

import jax
import jax.numpy as jnp
from jax.experimental import pallas as pl
from jax.experimental.pallas import tpu as pltpu

_ROWS = 16000
_CHUNK = 64
_NCHUNK_PAD = 256
_S_PAD = 264
_GATHER = 13


def _body(cidx_ref, rmod_ref, e_ref, xa_ref, xb_ref, *rest, num_blocks, rows, n_seg):
    g_refs = rest[:_GATHER]
    w1_ref, b1_ref, w2_ref, b2_ref, w3_ref, b3_ref, out_ref, acc_ref = \
        rest[_GATHER:]
    b = pl.program_id(0)

    @pl.when(b == 0)
    def _build():
        acc_ref[...] = jnp.zeros_like(acc_ref)

    half = rows // 2
    a1 = jnp.dot(xa_ref[...], w1_ref[...], preferred_element_type=jnp.float32)
    a1 = jnp.maximum(a1 + b1_ref[...], 0.0)
    a2 = jnp.dot(xb_ref[...], w1_ref[...], preferred_element_type=jnp.float32)
    a2 = jnp.maximum(a2 + b1_ref[...], 0.0)
    xg = jnp.concatenate([g[...] for g in g_refs], axis=0)
    ag = jnp.dot(xg, w1_ref[...], preferred_element_type=jnp.float32)
    ag = jnp.maximum(ag + b1_ref[...], 0.0)

    csum_1 = jnp.sum(a1.reshape(half // _CHUNK, _CHUNK, -1), axis=1)
    csum_2 = jnp.sum(a2.reshape(half // _CHUNK, _CHUNK, -1), axis=1)
    csum = jnp.concatenate(
        [csum_1, csum_2,
         jnp.zeros((_NCHUNK_PAD - rows // _CHUNK, csum_1.shape[1]),
                   jnp.float32)], axis=0)
    cstart = b * rows + _CHUNK * jax.lax.broadcasted_iota(
        jnp.int32, (1, _NCHUNK_PAD), 1)
    stair = (cstart >= e_ref[...]).astype(jnp.float32)
    acc_ref[...] += jnp.dot(stair, csum, preferred_element_type=jnp.float32)

    riota = jax.lax.broadcasted_iota(jnp.int32, (_CHUNK, 1), 0)
    corrs = []
    for k in range(_GATHER):
        t = jnp.clip(b * _GATHER + k, 0, rmod_ref.shape[0] - 1)
        rmod = rmod_ref[t]
        mask = (riota >= rmod) & (rmod > 0)
        rows_k = ag[k * _CHUNK:(k + 1) * _CHUNK, :]
        corrs.append(jnp.sum(jnp.where(mask, rows_k, 0.0), axis=0,
                             keepdims=True))
    start = jnp.minimum(b * _GATHER, _S_PAD - _GATHER)
    acc_ref[pl.ds(start, _GATHER), :] += jnp.concatenate(corrs, axis=0)

    @pl.when(b == num_blocks - 1)
    def _finalize():
        seg = acc_ref[0:n_seg, :] - acc_ref[1:n_seg + 1, :]
        d = e_ref[1:n_seg + 1, :] - e_ref[0:n_seg, :]
        counts = jnp.maximum(d.astype(jnp.float32), 1.0)
        mean = seg / counts
        h2 = jnp.dot(mean, w2_ref[...], preferred_element_type=jnp.float32) + b2_ref[...]
        out_ref[...] = jnp.dot(h2, w3_ref[...], preferred_element_type=jnp.float32) + b3_ref[...]


def kernel(x1, edge_slices, W1, b1, W2, b2, W3, b3):
    n, d_in = x1.shape
    d_out = W2.shape[1]
    n_seg = edge_slices.shape[0] - 1
    rows = _ROWS
    num_blocks = n // rows
    assert num_blocks * rows == n and rows % _CHUNK == 0

    n_bound = num_blocks * _GATHER
    e_pad = jnp.concatenate(
        [edge_slices,
         jnp.full((_S_PAD - edge_slices.shape[0],), n, dtype=jnp.int32)])
    e_gather = jnp.concatenate(
        [edge_slices,
         jnp.full((n_bound - edge_slices.shape[0],), n, dtype=jnp.int32)])
    chunk_idx = jnp.clip(e_gather // _CHUNK, 0, n // _CHUNK - 1).astype(jnp.int32)
    rmod = (e_gather % _CHUNK).astype(jnp.int32)

    body = functools.partial(_body, num_blocks=num_blocks, rows=rows,
                             n_seg=n_seg)
    full = lambda shape: pl.BlockSpec(shape, lambda b, ci, rm: (0, 0))

    def _gather_spec(k):
        def imap(b, ci, rm):
            t = jnp.clip(b * _GATHER + k, 0, n_bound - 1)
            return (ci[t], 0)
        return pl.BlockSpec((_CHUNK, d_in), imap)

    grid_spec = pltpu.PrefetchScalarGridSpec(
        num_scalar_prefetch=2,
        grid=(num_blocks,),
        in_specs=[
            full((_S_PAD, 1)),
            pl.BlockSpec((rows // 2, d_in), lambda b, ci, rm: (2 * b, 0)),
            pl.BlockSpec((rows // 2, d_in), lambda b, ci, rm: (2 * b + 1, 0)),
            *[_gather_spec(k) for k in range(_GATHER)],
            full((d_in, d_out)),
            full((1, d_out)),
            full((d_out, d_out)),
            full((1, d_out)),
            full((d_out, d_out)),
            full((1, d_out)),
        ],
        out_specs=full((n_seg, d_out)),
        scratch_shapes=[
            pltpu.VMEM((_S_PAD, d_out), jnp.float32),
        ],
    )
    out = pl.pallas_call(
        body,
        grid_spec=grid_spec,
        out_shape=jax.ShapeDtypeStruct((n_seg, d_out), jnp.float32),
        compiler_params=pltpu.CompilerParams(
            dimension_semantics=("arbitrary",),
        ),
    )(chunk_idx, rmod, e_pad.reshape(_S_PAD, 1), x1, x1,
      *([x1] * _GATHER), W1, b1.reshape(1, d_out), W2, b2.reshape(1, d_out),
      W3, b3.reshape(1, d_out))
    return out
